# baseline (speedup 1.0000x reference)
import jax
import jax.numpy as jnp
from jax.experimental import pallas as pl
from jax.experimental.pallas import tpu as pltpu

_B, _C, _H, _W = 64, 384, 14, 14
_E = 8
_K = 2
_HID = 2 * _C
_HW = _H * _W
_SB = 8


def _routing_kernel(x_ref, wr_ref, br_ref, idx_ref, wts_ref):
    xb = x_ref[...]
    pooled = jnp.mean(xb, axis=2)
    logits = jax.lax.dot_general(
        pooled, wr_ref[...], (((1,), (1,)), ((), ())),
        preferred_element_type=jnp.float32) + br_ref[...]
    probs = jax.nn.softmax(logits, axis=-1)
    lane = jax.lax.broadcasted_iota(jnp.int32, probs.shape, 1)
    a1 = jnp.argmax(probs, axis=-1)
    m1 = jnp.max(probs, axis=-1)
    masked = jnp.where(lane == a1[:, None], -jnp.inf, probs)
    a2 = jnp.argmax(masked, axis=-1)
    m2 = jnp.max(masked, axis=-1)
    denom = m1 + m2
    idx_ref[...] = jnp.concatenate([a1[:, None], a2[:, None]], axis=1)
    wts_ref[...] = jnp.concatenate(
        [(m1 / denom)[:, None], (m2 / denom)[:, None]], axis=1)


def _moe_kernel(idx_ref, wts_ref, x_ref, ws_ref, gamma_ref, beta_ref,
                w1_ref, w2_ref, out_ref):
    s = pl.program_id(0)
    xb = x_ref[0]
    xb16 = xb.astype(jnp.bfloat16)
    e0 = idx_ref[s, 0]
    e1 = idx_ref[s, 1]
    w0 = wts_ref[s, 0]
    w1 = wts_ref[s, 1]

    h0 = jnp.dot(w1_ref[e0], xb16, preferred_element_type=jnp.float32)
    h0 = h0 * jax.nn.sigmoid(h0)
    out0 = jnp.dot(w2_ref[e0], h0.astype(jnp.bfloat16),
                   preferred_element_type=jnp.float32)

    h1 = jnp.dot(w1_ref[e1], xb16, preferred_element_type=jnp.float32)
    h1 = h1 * jax.nn.sigmoid(h1)
    out1 = jnp.dot(w2_ref[e1], h1.astype(jnp.bfloat16),
                   preferred_element_type=jnp.float32)

    shared = jnp.dot(ws_ref[...], xb16, preferred_element_type=jnp.float32)
    shared = shared * gamma_ref[...] + beta_ref[...]
    shared = shared * jax.nn.sigmoid(shared)

    out_ref[0] = xb + shared + w0 * out0 + w1 * out1


def kernel(x, Wr, br, Ws, gamma, beta, W1, W2):
    xr = x.reshape(_B, _C, _HW)

    idx, wts = pl.pallas_call(
        _routing_kernel,
        grid=(_B // _SB,),
        in_specs=[
            pl.BlockSpec((_SB, _C, _HW), lambda i: (i, 0, 0)),
            pl.BlockSpec((_E, _C), lambda i: (0, 0)),
            pl.BlockSpec((1, _E), lambda i: (0, 0)),
        ],
        out_specs=[
            pl.BlockSpec((_SB, _K), lambda i: (i, 0)),
            pl.BlockSpec((_SB, _K), lambda i: (i, 0)),
        ],
        out_shape=[
            jax.ShapeDtypeStruct((_B, _K), jnp.int32),
            jax.ShapeDtypeStruct((_B, _K), jnp.float32),
        ],
    )(xr, Wr, br.reshape(1, _E))

    grid_spec = pltpu.PrefetchScalarGridSpec(
        num_scalar_prefetch=2,
        grid=(_B,),
        in_specs=[
            pl.BlockSpec((1, _C, _HW), lambda s, idx, wts: (s, 0, 0)),
            pl.BlockSpec((_C, _C), lambda s, idx, wts: (0, 0)),
            pl.BlockSpec((_C, 1), lambda s, idx, wts: (0, 0)),
            pl.BlockSpec((_C, 1), lambda s, idx, wts: (0, 0)),
            pl.BlockSpec((_E, _HID, _C), lambda s, idx, wts: (0, 0, 0)),
            pl.BlockSpec((_E, _C, _HID), lambda s, idx, wts: (0, 0, 0)),
        ],
        out_specs=pl.BlockSpec((1, _C, _HW), lambda s, idx, wts: (s, 0, 0)),
    )
    out = pl.pallas_call(
        _moe_kernel,
        grid_spec=grid_spec,
        out_shape=jax.ShapeDtypeStruct((_B, _C, _HW), jnp.float32),
        compiler_params=pltpu.CompilerParams(
            dimension_semantics=("arbitrary",)),
    )(idx, wts, xr, Ws.astype(jnp.bfloat16), gamma.reshape(_C, 1),
      beta.reshape(_C, 1), W1.astype(jnp.bfloat16), W2.astype(jnp.bfloat16))

    return out.reshape(_B, _C, _H, _W)
